# baseline (speedup 1.0000x reference)
import jax
import jax.numpy as jnp
from jax.experimental import pallas as pl
from jax.experimental.pallas import tpu as pltpu

_TH = 16


def _body(img_ref, w_ref, p_ref, o_ref):
    m = jnp.where(p_ref[...], 0.0, w_ref[...])
    o_ref[...] = img_ref[...] * m[None, None, :, :]


def kernel(image, weight, pruned):
    B, C, H, W = image.shape
    grid = (H // _TH,)
    out = pl.pallas_call(
        _body,
        grid=grid,
        in_specs=[
            pl.BlockSpec((B, C, _TH, W), lambda i: (0, 0, i, 0)),
            pl.BlockSpec((_TH, W), lambda i: (i, 0)),
            pl.BlockSpec((_TH, W), lambda i: (i, 0)),
        ],
        out_specs=pl.BlockSpec((B, C, _TH, W), lambda i: (0, 0, i, 0)),
        out_shape=jax.ShapeDtypeStruct((B, C, H, W), image.dtype),
        compiler_params=pltpu.CompilerParams(
            dimension_semantics=("parallel",),
        ),
    )(image, weight, pruned)
    return out[None]


# speedup vs baseline: 1.0112x; 1.0112x over previous
import jax
import jax.numpy as jnp
from jax.experimental import pallas as pl
from jax.experimental.pallas import tpu as pltpu

_TB = 4


def _body(img_ref, w_ref, p_ref, o_ref):
    m = jnp.where(p_ref[...], 0.0, w_ref[...])
    o_ref[...] = img_ref[...] * m[None, :, :]


def kernel(image, weight, pruned):
    B, C, H, W = image.shape
    BC = B * C
    img = image.reshape(BC, H, W)
    out = pl.pallas_call(
        _body,
        grid=(BC // _TB,),
        in_specs=[
            pl.BlockSpec((_TB, H, W), lambda i: (i, 0, 0)),
            pl.BlockSpec((H, W), lambda i: (0, 0)),
            pl.BlockSpec((H, W), lambda i: (0, 0)),
        ],
        out_specs=pl.BlockSpec((_TB, H, W), lambda i: (i, 0, 0)),
        out_shape=jax.ShapeDtypeStruct((BC, H, W), image.dtype),
        compiler_params=pltpu.CompilerParams(
            dimension_semantics=("arbitrary",),
        ),
    )(img, weight, pruned)
    return out.reshape(1, B, C, H, W)


# speedup vs baseline: 1.0346x; 1.0232x over previous
import jax
import jax.numpy as jnp
from jax.experimental import pallas as pl
from jax.experimental.pallas import tpu as pltpu

_TB = 8


def _body(img_ref, w_ref, p_ref, o_ref):
    m = jnp.where(p_ref[...], 0.0, w_ref[...])
    o_ref[...] = img_ref[...] * m[None, :, :]


def kernel(image, weight, pruned):
    B, C, H, W = image.shape
    BC = B * C
    img = image.reshape(BC, H, W)
    out = pl.pallas_call(
        _body,
        grid=(BC // _TB,),
        in_specs=[
            pl.BlockSpec((_TB, H, W), lambda i: (i, 0, 0)),
            pl.BlockSpec((H, W), lambda i: (0, 0)),
            pl.BlockSpec((H, W), lambda i: (0, 0)),
        ],
        out_specs=pl.BlockSpec((_TB, H, W), lambda i: (i, 0, 0)),
        out_shape=jax.ShapeDtypeStruct((BC, H, W), image.dtype),
        compiler_params=pltpu.CompilerParams(
            dimension_semantics=("arbitrary",),
        ),
    )(img, weight, pruned)
    return out.reshape(1, B, C, H, W)
